# baseline (speedup 1.0000x reference)
import jax
import jax.numpy as jnp
from jax import lax
from jax.experimental import pallas as pl
from jax.experimental.pallas import tpu as pltpu
from jax.experimental.pallas import tpu_sc as plsc

N = 32768
B = 16
C = 96
NC = 2
NS = 16
NW = NC * NS
FR = 8
CW = 4096
NO = N // CW
TPW = (C // FR) * NO // NW
EPS = 1e-5


def _sc_body(featT_hbm, cu_hbm, out_hbm, buf0, buf1, cu_v, acc_v,
             sem0, sem1):
    wid = lax.axis_index("s") * NC + lax.axis_index("c")
    t0 = wid * TPW
    bufs = (buf0, buf1)
    sems = (sem0, sem1)

    def task_slices(t):
        g = lax.div(t, NO)
        o = lax.rem(t, NO)
        r0 = pl.multiple_of(g * FR, FR)
        cb = pl.multiple_of(o * CW, CW)
        return r0, cb, o

    def start(j):
        t = t0 + j
        r0, cb, _ = task_slices(t)
        return pltpu.async_copy(
            featT_hbm.at[pl.ds(r0, FR), pl.ds(cb, CW)],
            bufs[j % 2], sems[j % 2])

    copies = [start(0)]
    pltpu.sync_copy(cu_hbm, cu_v)
    cu_a = cu_v[pl.ds(0, 16)]
    cu_b = cu_v[pl.ds(16, 16)]
    cu_s = [cu_a[j] if j < 16 else cu_b[j - 16] for j in range(B + 1)]
    lane = lax.iota(jnp.int32, 16)
    zero = jnp.zeros((16,), jnp.float32)
    for j in range(TPW):
        if j + 1 < TPW:
            copies.append(start(j + 1))
        copies[j].wait()
        buf = bufs[j % 2]
        t = t0 + j
        r0, cb, o = task_slices(t)
        c0 = cb
        c1 = cb + CW
        for b in range(B):
            s = jnp.maximum(cu_s[b], c0)
            e = jnp.minimum(cu_s[b + 1], c1)
            bi0 = lax.shift_right_logical(s + 15, 4)
            bi1 = jnp.maximum(bi0, lax.shift_right_logical(e, 4))
            hb = jnp.bitwise_and(jnp.minimum(s, c1 - 1), -16)
            la = hb + lane
            mh = (la >= s) & (la < jnp.minimum(e, bi0 * 16))
            hoff = pl.multiple_of(hb - c0, 16)
            segs = [jnp.where(mh, buf[f, pl.ds(hoff, 16)], zero)
                    for f in range(FR)]

            def body(off, accs):
                o16 = pl.multiple_of(off, 16)
                return tuple(accs[f] + buf[f, pl.ds(o16, 16)]
                             for f in range(FR))

            segs = list(plsc.parallel_loop(
                bi0 * 16 - c0, bi1 * 16 - c0, 16, unroll=1,
                carry=tuple(segs))(body))
            tb = jnp.bitwise_and(e, -16)
            tbc = jnp.minimum(jnp.maximum(tb, c0), c1 - 16)
            lt = tbc + lane
            mt = (lt >= jnp.maximum(bi0 * 16, tb)) & (lt < e)
            toff = pl.multiple_of(tbc - c0, 16)
            for f in range(FR):
                seg = segs[f] + jnp.where(mt, buf[f, pl.ds(toff, 16)],
                                          zero)
                acc_v[f, pl.ds(b * 16, 16)] = seg
        pltpu.sync_copy(acc_v, out_hbm.at[o, pl.ds(r0, FR)])


def _sc_seg_sums(featT, cu_pad):
    return pl.kernel(
        _sc_body,
        out_type=jax.ShapeDtypeStruct((NO, C, B * 16), jnp.float32),
        mesh=plsc.VectorSubcoreMesh(core_axis_name="c", subcore_axis_name="s"),
        scratch_types=[
            pltpu.VMEM((FR, CW), jnp.float32),
            pltpu.VMEM((FR, CW), jnp.float32),
            pltpu.VMEM((32,), jnp.int32),
            pltpu.VMEM((FR, B * 16), jnp.float32),
            pltpu.SemaphoreType.DMA,
            pltpu.SemaphoreType.DMA,
        ],
    )(featT, cu_pad)


def _head_body(sums_ref, lo_ref, hi_ref, bracket_ref,
               W1_ref, b1_ref, g1_ref, be1_ref, m1_ref, v1_ref,
               W2_ref, b2_ref, g2_ref, be2_ref, m2_ref, v2_ref,
               W3_ref, b3_ref,
               pred_ref, loss_ref, cos_ref):
    lo = lo_ref[...]
    hi = hi_ref[...]
    counts = jnp.maximum((hi - lo).astype(jnp.float32), 1.0)
    sums2 = jnp.sum(sums_ref[...], axis=0)
    grp = (lax.broadcasted_iota(jnp.int32, (16 * B, B), 0) // 16
           == lax.broadcasted_iota(jnp.int32, (16 * B, B), 1))
    sums = jnp.dot(sums2, grp.astype(jnp.float32),
                   preferred_element_type=jnp.float32,
                   precision=lax.Precision.HIGHEST)
    pooledT = sums / counts
    h = lax.dot_general(pooledT, W1_ref[...],
                        dimension_numbers=(((0,), (0,)), ((), ())),
                        preferred_element_type=jnp.float32)
    h = h + b1_ref[...]
    h = g1_ref[...] * (h - m1_ref[...]) * lax.rsqrt(v1_ref[...] + EPS) \
        + be1_ref[...]
    h = jnp.maximum(h, 0.0)
    h = jnp.dot(h, W2_ref[...], preferred_element_type=jnp.float32)
    h = h + b2_ref[...]
    h = g2_ref[...] * (h - m2_ref[...]) * lax.rsqrt(v2_ref[...] + EPS) \
        + be2_ref[...]
    h = jnp.maximum(h, 0.0)
    pred = jnp.dot(h, W3_ref[...], preferred_element_type=jnp.float32)
    pred = pred + b3_ref[...]
    pred_ref[...] = pred
    target = bracket_ref[...]
    diff = pred - target
    loss_ref[...] = jnp.mean(diff * diff).reshape(1, 1)
    num = jnp.sum(pred * target, axis=1)
    den = (jnp.maximum(jnp.sqrt(jnp.sum(pred * pred, axis=1)), 1e-8)
           * jnp.maximum(jnp.sqrt(jnp.sum(target * target, axis=1)), 1e-8))
    cos_ref[...] = jnp.mean(num / den).reshape(1, 1)


def kernel(feat, cu_seqlens, bracket, W1, b1, g1, be1, m1, v1,
           W2, b2, g2, be2, m2, v2, W3, b3):
    cu_pad = jnp.concatenate(
        [cu_seqlens, jnp.zeros((32 - (B + 1),), jnp.int32)])
    sums = _sc_seg_sums(feat.T, cu_pad)

    lo = cu_seqlens[:-1].reshape(1, B)
    hi = cu_seqlens[1:].reshape(1, B)

    pred, loss, cos = pl.pallas_call(
        _head_body,
        out_shape=[
            jax.ShapeDtypeStruct((B, 3), jnp.float32),
            jax.ShapeDtypeStruct((1, 1), jnp.float32),
            jax.ShapeDtypeStruct((1, 1), jnp.float32),
        ],
    )(sums, lo, hi, bracket,
      W1, b1.reshape(1, 256), g1.reshape(1, 256), be1.reshape(1, 256),
      m1.reshape(1, 256), v1.reshape(1, 256),
      W2, b2.reshape(1, 128), g2.reshape(1, 128), be2.reshape(1, 128),
      m2.reshape(1, 128), v2.reshape(1, 128),
      W3, b3.reshape(1, 3))
    return (pred, loss[0, 0], cos[0, 0])


# speedup vs baseline: 1.1861x; 1.1861x over previous
import jax
import jax.numpy as jnp
from jax import lax
from jax.experimental import pallas as pl
from jax.experimental.pallas import tpu as pltpu
from jax.experimental.pallas import tpu_sc as plsc

N = 32768
B = 16
C = 96
CT = 64
CS = C - CT
NC = 2
NS = 16
NW = NC * NS
FR = 8
CW = 4096
NO = N // CW
RB = 4096
NBLK = N // RB
EPS = 1e-5


def _sc_body(featT_hbm, cu_hbm, out_hbm, buf, cu_v, acc_v, sem0):
    wid = lax.axis_index("s") * NC + lax.axis_index("c")
    t = CT * NO // FR + wid
    g = lax.div(t, NO)
    o = lax.rem(t, NO)
    r0 = pl.multiple_of(g * FR, FR)
    cb = pl.multiple_of(o * CW, CW)
    cp = pltpu.async_copy(
        featT_hbm.at[pl.ds(r0, FR), pl.ds(cb, CW)], buf, sem0)
    pltpu.sync_copy(cu_hbm, cu_v)
    cu_a = cu_v[pl.ds(0, 16)]
    cu_b = cu_v[pl.ds(16, 16)]
    cu_s = [cu_a[j] if j < 16 else cu_b[j - 16] for j in range(B + 1)]
    lane = lax.iota(jnp.int32, 16)
    zero = jnp.zeros((16,), jnp.float32)
    cp.wait()
    c0 = cb
    c1 = cb + CW
    for b in range(B):
        s = jnp.maximum(cu_s[b], c0)
        e = jnp.minimum(cu_s[b + 1], c1)
        bi0 = lax.shift_right_logical(s + 15, 4)
        bi1 = jnp.maximum(bi0, lax.shift_right_logical(e, 4))
        hb = jnp.bitwise_and(jnp.minimum(s, c1 - 1), -16)
        la = hb + lane
        mh = (la >= s) & (la < jnp.minimum(e, bi0 * 16))
        hoff = pl.multiple_of(hb - c0, 16)
        segs = [jnp.where(mh, buf[f, pl.ds(hoff, 16)], zero)
                for f in range(FR)]

        def body(off, accs):
            o16 = pl.multiple_of(off, 16)
            return tuple(accs[f] + buf[f, pl.ds(o16, 16)]
                         for f in range(FR))

        segs = list(plsc.parallel_loop(
            bi0 * 16 - c0, bi1 * 16 - c0, 16, unroll=1,
            carry=tuple(segs))(body))
        tb = jnp.bitwise_and(e, -16)
        tbc = jnp.minimum(jnp.maximum(tb, c0), c1 - 16)
        lt = tbc + lane
        mt = (lt >= jnp.maximum(bi0 * 16, tb)) & (lt < e)
        toff = pl.multiple_of(tbc - c0, 16)
        for f in range(FR):
            seg = segs[f] + jnp.where(mt, buf[f, pl.ds(toff, 16)], zero)
            acc_v[f, pl.ds(b * 16, 16)] = seg
    ro = pl.multiple_of(r0 - CT, FR)
    pltpu.sync_copy(acc_v, out_hbm.at[o, pl.ds(ro, FR)])


def _sc_seg_sums(featT, cu_pad):
    return pl.kernel(
        _sc_body,
        out_type=jax.ShapeDtypeStruct((NO, CS, B * 16), jnp.float32),
        mesh=plsc.VectorSubcoreMesh(core_axis_name="c", subcore_axis_name="s"),
        scratch_types=[
            pltpu.VMEM((FR, CW), jnp.float32),
            pltpu.VMEM((32,), jnp.int32),
            pltpu.VMEM((FR, B * 16), jnp.float32),
            pltpu.SemaphoreType.DMA,
        ],
    )(featT, cu_pad)


def _top_body(lo_ref, hi_ref, featT_ref, sums_ref, acc_ref):
    i = pl.program_id(0)

    @pl.when(i == 0)
    def _():
        acc_ref[...] = jnp.zeros_like(acc_ref)

    idx = lax.broadcasted_iota(jnp.int32, (RB, B), 0) + i * RB
    mask = ((idx >= lo_ref[...]) & (idx < hi_ref[...])).astype(jnp.float32)
    acc_ref[...] += jnp.dot(featT_ref[...], mask,
                            preferred_element_type=jnp.float32,
                            precision=lax.Precision.HIGHEST)

    @pl.when(i == NBLK - 1)
    def _():
        sums_ref[...] = acc_ref[...]


def _head_body(sums_top_ref, part_ref, lo_ref, hi_ref, bracket_ref,
               W1_ref, b1_ref, g1_ref, be1_ref, m1_ref, v1_ref,
               W2_ref, b2_ref, g2_ref, be2_ref, m2_ref, v2_ref,
               W3_ref, b3_ref,
               pred_ref, loss_ref, cos_ref):
    lo = lo_ref[...]
    hi = hi_ref[...]
    counts = jnp.maximum((hi - lo).astype(jnp.float32), 1.0)
    bot = jnp.sum(part_ref[...], axis=0)
    grp = (lax.broadcasted_iota(jnp.int32, (16 * B, B), 0) // 16
           == lax.broadcasted_iota(jnp.int32, (16 * B, B), 1))
    sums_bot = jnp.dot(bot, grp.astype(jnp.float32),
                       preferred_element_type=jnp.float32,
                       precision=lax.Precision.HIGHEST)
    sumsT = jnp.concatenate([sums_top_ref[...], sums_bot], axis=0)
    pooledT = sumsT / counts
    h = lax.dot_general(pooledT, W1_ref[...],
                        dimension_numbers=(((0,), (0,)), ((), ())),
                        preferred_element_type=jnp.float32)
    h = h + b1_ref[...]
    h = g1_ref[...] * (h - m1_ref[...]) * lax.rsqrt(v1_ref[...] + EPS) \
        + be1_ref[...]
    h = jnp.maximum(h, 0.0)
    h = jnp.dot(h, W2_ref[...], preferred_element_type=jnp.float32)
    h = h + b2_ref[...]
    h = g2_ref[...] * (h - m2_ref[...]) * lax.rsqrt(v2_ref[...] + EPS) \
        + be2_ref[...]
    h = jnp.maximum(h, 0.0)
    pred = jnp.dot(h, W3_ref[...], preferred_element_type=jnp.float32)
    pred = pred + b3_ref[...]
    pred_ref[...] = pred
    target = bracket_ref[...]
    diff = pred - target
    loss_ref[...] = jnp.mean(diff * diff).reshape(1, 1)
    num = jnp.sum(pred * target, axis=1)
    den = (jnp.maximum(jnp.sqrt(jnp.sum(pred * pred, axis=1)), 1e-8)
           * jnp.maximum(jnp.sqrt(jnp.sum(target * target, axis=1)), 1e-8))
    cos_ref[...] = jnp.mean(num / den).reshape(1, 1)


def kernel(feat, cu_seqlens, bracket, W1, b1, g1, be1, m1, v1,
           W2, b2, g2, be2, m2, v2, W3, b3):
    featT = feat.T
    cu_pad = jnp.concatenate(
        [cu_seqlens, jnp.zeros((32 - (B + 1),), jnp.int32)])
    partials = _sc_seg_sums(featT, cu_pad)

    lo = cu_seqlens[:-1].reshape(1, B)
    hi = cu_seqlens[1:].reshape(1, B)

    def whole(shape):
        return pl.BlockSpec(shape, lambda i: (0,) * len(shape))

    sums_top = pl.pallas_call(
        _top_body,
        grid_spec=pltpu.PrefetchScalarGridSpec(
            num_scalar_prefetch=0,
            grid=(NBLK,),
            in_specs=[
                whole((1, B)),
                whole((1, B)),
                pl.BlockSpec((CT, RB), lambda i: (0, i)),
            ],
            out_specs=[whole((CT, B))],
            scratch_shapes=[pltpu.VMEM((CT, B), jnp.float32)],
        ),
        out_shape=[jax.ShapeDtypeStruct((CT, B), jnp.float32)],
    )(lo, hi, featT)[0]

    pred, loss, cos = pl.pallas_call(
        _head_body,
        out_shape=[
            jax.ShapeDtypeStruct((B, 3), jnp.float32),
            jax.ShapeDtypeStruct((1, 1), jnp.float32),
            jax.ShapeDtypeStruct((1, 1), jnp.float32),
        ],
    )(sums_top, partials, lo, hi, bracket,
      W1, b1.reshape(1, 256), g1.reshape(1, 256), be1.reshape(1, 256),
      m1.reshape(1, 256), v1.reshape(1, 256),
      W2, b2.reshape(1, 128), g2.reshape(1, 128), be2.reshape(1, 128),
      m2.reshape(1, 128), v2.reshape(1, 128),
      W3, b3.reshape(1, 3))
    return (pred, loss[0, 0], cos[0, 0])
